# baseline (speedup 1.0000x reference)
import functools

import jax
import jax.numpy as jnp
from jax import lax
from jax.experimental import pallas as pl
from jax.experimental.pallas import tpu as pltpu
from jax.experimental.pallas import tpu_sc as plsc

NUM_FACTORS = 16
BATCH = 16384
LANES = 16
CHUNK = 128

_info = plsc.get_sparse_core_info()
_NC, _NS = _info.num_cores, _info.num_subcores
_NW = _NC * _NS
_BPW = BATCH // _NW
_NCHUNK = _BPW // CHUNK
_NBLK = _BPW // LANES


def _mf_body(user_hbm, movie_hbm, uf_hbm, mf_hbm, out_hbm,
             uidx_v, midx_v, urows_v, mrows_v, out_v, sem):
    wid = lax.axis_index("s") * _NC + lax.axis_index("c")
    base = wid * _BPW

    for j in range(_NCHUNK):
        pltpu.sync_copy(user_hbm.at[pl.ds(base + j * CHUNK, CHUNK)], uidx_v.at[j])
        pltpu.sync_copy(movie_hbm.at[pl.ds(base + j * CHUNK, CHUNK)], midx_v.at[j])

    descs = []
    for j in range(_NCHUNK):
        descs.append(pltpu.async_copy(
            uf_hbm.at[uidx_v.at[j]], urows_v.at[pl.ds(j * CHUNK, CHUNK), :], sem))
        descs.append(pltpu.async_copy(
            mf_hbm.at[midx_v.at[j]], mrows_v.at[pl.ds(j * CHUNK, CHUNK), :], sem))
    for d in descs:
        d.wait()

    iota = lax.iota(jnp.int32, LANES)
    cols = [jnp.full((LANES,), f, jnp.int32) for f in range(NUM_FACTORS)]

    def blk_body(blk, carry):
        rows = blk * LANES + iota
        acc = jnp.zeros((LANES,), jnp.float32)
        for f in range(NUM_FACTORS):
            uv = plsc.load_gather(urows_v, [rows, cols[f]])
            mv = plsc.load_gather(mrows_v, [rows, cols[f]])
            acc = acc + uv * mv
        out_v[pl.ds(blk * LANES, LANES)] = acc
        return carry

    lax.fori_loop(0, _NBLK, blk_body, 0)

    pltpu.sync_copy(out_v, out_hbm.at[pl.ds(base, _BPW)])


_mf_kernel = functools.partial(
    pl.kernel,
    out_type=jax.ShapeDtypeStruct((BATCH,), jnp.float32),
    mesh=plsc.VectorSubcoreMesh(core_axis_name="c", subcore_axis_name="s"),
    compiler_params=pltpu.CompilerParams(
        needs_layout_passes=False, use_tc_tiling_on_sc=False),
    scratch_types=[
        pltpu.VMEM((_NCHUNK, CHUNK), jnp.int32),
        pltpu.VMEM((_NCHUNK, CHUNK), jnp.int32),
        pltpu.VMEM((_BPW, NUM_FACTORS), jnp.float32),
        pltpu.VMEM((_BPW, NUM_FACTORS), jnp.float32),
        pltpu.VMEM((_BPW,), jnp.float32),
        pltpu.SemaphoreType.DMA,
    ],
)(_mf_body)


def kernel(user, movie, user_factors, movie_factors):
    return _mf_kernel(user.astype(jnp.int32), movie.astype(jnp.int32),
                      user_factors, movie_factors)


# speedup vs baseline: 1.7139x; 1.7139x over previous
import functools

import jax
import jax.numpy as jnp
from jax import lax
from jax.experimental import pallas as pl
from jax.experimental.pallas import tpu as pltpu
from jax.experimental.pallas import tpu_sc as plsc

F = 16
B = 16384
L = 16
NUSERS = 1000000

_info = plsc.get_sparse_core_info()
_NC, _NS = _info.num_cores, _info.num_subcores
_NW = _NC * _NS
_BPW = B // _NW

W = 1024
_FULL = NUSERS // W
_TAILLO = _FULL * W
_TAILW = 640
_NP = -(-(_FULL + 1) // _NW)
_NVEC = B // L
_CAP = B
_LCAP = B + 2 * L


def _stage_a(user_hbm, uft_hbm, vals_hbm, slots_hbm, counts_hbm,
             uid_v, ls_v, lu_v, chunk_a, chunk_b, tail_v, pend_v, pslot_v,
             stag0, stag1, sstag0, sstag1, junk_v, junks_v,
             sem, sem2, fs0, fs1):
    wid = lax.axis_index("s") * _NC + lax.axis_index("c")

    pltpu.sync_copy(user_hbm, uid_v)

    iota = lax.iota(jnp.int32, L)
    cols = [jnp.full((L,), f, jnp.int32) for f in range(F)]
    vbase = wid * _CAP * F
    sbase = wid * _CAP

    def sweep(v, cur):
        u = uid_v[pl.ds(v * L, L)]
        mine = (lax.shift_right_logical(u, 10) & 31) == wid
        plsc.store_compressed(ls_v.at[pl.ds(cur, L)], v * L + iota, mask=mine)
        plsc.store_compressed(lu_v.at[pl.ds(cur, L)], u, mask=mine)
        return cur + plsc.all_reduce_population_count(mine)[0]

    cur = lax.fori_loop(0, _NVEC, sweep, jnp.int32(0))

    ls_v[pl.ds(cur, L)] = jnp.full((L,), -1, jnp.int32)
    lu_v[pl.ds(cur, L)] = jnp.zeros((L,), jnp.int32)
    lsgrp = lax.shift_right_logical(cur + L - 1, 4)

    def emit(ngrp, stag, sstag, fsem):
        @pl.when(ngrp >= 2)
        def _():
            pltpu.make_async_copy(
                vals_hbm.at[pl.ds(0, F * L)], junk_v, fsem).wait()
            pltpu.make_async_copy(
                slots_hbm.at[pl.ds(0, L)], junks_v, fsem).wait()
        for f in range(F):
            col = pend_v[f, pl.ds(0, L)]
            plsc.store_scatter(stag, [iota * F + f], col)
        sstag[pl.ds(0, L)] = pslot_v[pl.ds(0, L)]
        pltpu.async_copy(
            stag, vals_hbm.at[pl.ds(vbase + ngrp * L * F, L * F)], fsem)
        pltpu.async_copy(
            sstag, slots_hbm.at[pl.ds(sbase + ngrp * L, L)], fsem)

    def emit_either(ngrp):
        @pl.when(lax.rem(ngrp, 2) == 0)
        def _():
            emit(ngrp, stag0, sstag0, fs0)

        @pl.when(lax.rem(ngrp, 2) == 1)
        def _():
            emit(ngrp, stag1, sstag1, fs1)

    def extract(g, buf, width, carry0):
        def grp(i, carry):
            nf, ngrp = carry
            sl = ls_v[pl.ds(i * L, L)]
            u = lu_v[pl.ds(i * L, L)]
            m = lax.shift_right_logical(u, 10) == g
            n = plsc.all_reduce_population_count(m)[0]

            @pl.when(n > 0)
            def _():
                jloc = jnp.clip(u - g * W, 0, width - 1)
                for f in range(F):
                    vals = plsc.load_gather(buf, [cols[f], jloc])
                    plsc.store_compressed(
                        pend_v.at[f, pl.ds(nf, L)], vals, mask=m)
                plsc.store_compressed(pslot_v.at[pl.ds(nf, L)], sl, mask=m)

            nf2 = jnp.where(n > 0, nf + n, nf)
            full = nf2 >= L

            @pl.when(full)
            def _():
                emit_either(ngrp)
                for f in range(F):
                    t = pend_v[f, pl.ds(L, L)]
                    pend_v[f, pl.ds(0, L)] = t
                t = pslot_v[pl.ds(L, L)]
                pslot_v[pl.ds(0, L)] = t

            nf3 = jnp.where(full, nf2 - L, nf2)
            ngrp2 = jnp.where(full, ngrp + 1, ngrp)
            return (nf3, ngrp2)

        return lax.fori_loop(0, lsgrp, grp, carry0)

    def dma_pass(pp, buf, csem):
        g = jnp.minimum(pp * _NW + wid, _FULL - 1)
        lo = pl.multiple_of(g * W, 128)
        pltpu.async_copy(uft_hbm.at[:, pl.ds(lo, W)], buf, csem)

    def wait_pass(buf, csem):
        pltpu.make_async_copy(uft_hbm.at[:, pl.ds(0, W)], buf, csem).wait()

    dma_pass(0, chunk_a, sem)

    def pass_body(pp, carry):
        par = lax.rem(pp, 2)

        @pl.when(par == 0)
        def _():
            dma_pass(pp + 1, chunk_b, sem2)
            wait_pass(chunk_a, sem)

        @pl.when(par == 1)
        def _():
            dma_pass(pp + 1, chunk_a, sem)
            wait_pass(chunk_b, sem2)

        g = pp * _NW + wid
        c2 = extract(jnp.where(par == 0, g, -1), chunk_a, W, carry)
        c3 = extract(jnp.where(par == 1, g, -1), chunk_b, W, c2)
        return c3

    carry = lax.fori_loop(0, _NP - 1, pass_body, (jnp.int32(0), jnp.int32(0)))
    last_par = (_NP - 1) % 2
    if last_par == 0:
        wait_pass(chunk_a, sem)
    else:
        wait_pass(chunk_b, sem2)

    gl = (_NP - 1) * _NW + wid
    is_tail = gl == _FULL
    is_full = gl < _FULL
    lastbuf = (chunk_a, chunk_b)[(_NP - 1) % 2]

    @pl.when(is_tail)
    def _():
        pltpu.async_copy(
            uft_hbm.at[:, pl.ds(pl.multiple_of(_TAILLO, 128), _TAILW)],
            tail_v, sem).wait()

    carry = extract(jnp.where(is_full, gl, -1), lastbuf, W, carry)
    carry = extract(jnp.where(is_tail, gl, -1), tail_v, _TAILW, carry)
    nf, ngrp = carry

    @pl.when(nf > 0)
    def _():
        t = pslot_v[pl.ds(0, L)]
        pslot_v[pl.ds(0, L)] = jnp.where(iota < nf, t, -1)
        emit_either(ngrp)

    ngrp = jnp.where(nf > 0, ngrp + 1, ngrp)

    @pl.when(ngrp >= 1)
    def _():
        @pl.when(lax.rem(ngrp - 1, 2) == 0)
        def _():
            pltpu.make_async_copy(
                vals_hbm.at[pl.ds(0, F * L)], junk_v, fs0).wait()
            pltpu.make_async_copy(
                slots_hbm.at[pl.ds(0, L)], junks_v, fs0).wait()

        @pl.when(lax.rem(ngrp - 1, 2) == 1)
        def _():
            pltpu.make_async_copy(
                vals_hbm.at[pl.ds(0, F * L)], junk_v, fs1).wait()
            pltpu.make_async_copy(
                slots_hbm.at[pl.ds(0, L)], junks_v, fs1).wait()

    @pl.when(ngrp >= 2)
    def _():
        @pl.when(lax.rem(ngrp, 2) == 0)
        def _():
            pltpu.make_async_copy(
                vals_hbm.at[pl.ds(0, F * L)], junk_v, fs0).wait()
            pltpu.make_async_copy(
                slots_hbm.at[pl.ds(0, L)], junks_v, fs0).wait()

        @pl.when(lax.rem(ngrp, 2) == 1)
        def _():
            pltpu.make_async_copy(
                vals_hbm.at[pl.ds(0, F * L)], junk_v, fs1).wait()
            pltpu.make_async_copy(
                slots_hbm.at[pl.ds(0, L)], junks_v, fs1).wait()

    junks_v[pl.ds(0, L)] = jnp.full((L,), ngrp, jnp.int32)
    pltpu.sync_copy(junks_v, counts_hbm.at[pl.ds(wid * L, L)])


_stage_a_kernel = functools.partial(
    pl.kernel,
    out_type=(
        jax.ShapeDtypeStruct((_NW * _CAP * F,), jnp.float32),
        jax.ShapeDtypeStruct((_NW * _CAP,), jnp.int32),
        jax.ShapeDtypeStruct((_NW * L,), jnp.int32),
    ),
    mesh=plsc.VectorSubcoreMesh(core_axis_name="c", subcore_axis_name="s"),
    compiler_params=pltpu.CompilerParams(needs_layout_passes=False),
    scratch_types=[
        pltpu.VMEM((B,), jnp.int32),
        pltpu.VMEM((_LCAP,), jnp.int32),
        pltpu.VMEM((_LCAP,), jnp.int32),
        pltpu.VMEM((F, W), jnp.float32),
        pltpu.VMEM((F, W), jnp.float32),
        pltpu.VMEM((F, _TAILW), jnp.float32),
        pltpu.VMEM((F, 2 * L), jnp.float32),
        pltpu.VMEM((2 * L,), jnp.int32),
        pltpu.VMEM((L * F,), jnp.float32),
        pltpu.VMEM((L * F,), jnp.float32),
        pltpu.VMEM((L,), jnp.int32),
        pltpu.VMEM((L,), jnp.int32),
        pltpu.VMEM((L * F,), jnp.float32),
        pltpu.VMEM((L,), jnp.int32),
        pltpu.SemaphoreType.DMA,
        pltpu.SemaphoreType.DMA,
        pltpu.SemaphoreType.DMA,
        pltpu.SemaphoreType.DMA,
    ],
)(_stage_a)


def _stage_b0(slots_hbm, counts_hbm, pos_hbm,
              slots_v, cnt_v, prow0, prow1, sidx0, sidx1, junk_v, fs0, fs1):
    wid = lax.axis_index("s") * _NC + lax.axis_index("c")

    pltpu.sync_copy(counts_hbm.at[pl.ds(wid * L, L)], cnt_v)
    ngrp = cnt_v[pl.ds(0, L)][0]
    pltpu.sync_copy(slots_hbm.at[pl.ds(wid * _CAP, _CAP)], slots_v)

    iota = lax.iota(jnp.int32, L)
    cols = [jnp.full((L,), f, jnp.int32) for f in range(F)]

    def fire(g, prow, sidx, fsem):
        @pl.when(g >= 2)
        def _():
            pltpu.make_async_copy(
                pos_hbm.at[pl.ds(0, L), :], junk_v, fsem).wait()
        sl = slots_v[pl.ds(g * L, L)]
        pos = wid * _CAP + g * L + iota
        for f in range(F):
            plsc.store_scatter(prow, [iota, cols[f]], pos)
        sidx[pl.ds(0, L)] = jnp.where(sl >= 0, sl, B)
        pltpu.async_copy(prow, pos_hbm.at[sidx], fsem)

    def grp(g, c):
        @pl.when(lax.rem(g, 2) == 0)
        def _():
            fire(g, prow0, sidx0, fs0)

        @pl.when(lax.rem(g, 2) == 1)
        def _():
            fire(g, prow1, sidx1, fs1)
        return c

    lax.fori_loop(0, ngrp, grp, 0)

    @pl.when(ngrp >= 1)
    def _():
        @pl.when(lax.rem(ngrp - 1, 2) == 0)
        def _():
            pltpu.make_async_copy(
                pos_hbm.at[pl.ds(0, L), :], junk_v, fs0).wait()

        @pl.when(lax.rem(ngrp - 1, 2) == 1)
        def _():
            pltpu.make_async_copy(
                pos_hbm.at[pl.ds(0, L), :], junk_v, fs1).wait()

    @pl.when(ngrp >= 2)
    def _():
        @pl.when(lax.rem(ngrp, 2) == 0)
        def _():
            pltpu.make_async_copy(
                pos_hbm.at[pl.ds(0, L), :], junk_v, fs0).wait()

        @pl.when(lax.rem(ngrp, 2) == 1)
        def _():
            pltpu.make_async_copy(
                pos_hbm.at[pl.ds(0, L), :], junk_v, fs1).wait()


_stage_b0_kernel = functools.partial(
    pl.kernel,
    out_type=jax.ShapeDtypeStruct((B + 1, F), jnp.int32),
    mesh=plsc.VectorSubcoreMesh(core_axis_name="c", subcore_axis_name="s"),
    compiler_params=pltpu.CompilerParams(
        needs_layout_passes=False, use_tc_tiling_on_sc=False),
    scratch_types=[
        pltpu.VMEM((_CAP,), jnp.int32),
        pltpu.VMEM((L,), jnp.int32),
        pltpu.VMEM((L, F), jnp.int32),
        pltpu.VMEM((L, F), jnp.int32),
        pltpu.VMEM((L,), jnp.int32),
        pltpu.VMEM((L,), jnp.int32),
        pltpu.VMEM((L, F), jnp.int32),
        pltpu.SemaphoreType.DMA,
        pltpu.SemaphoreType.DMA,
    ],
)(_stage_b0)


def _stage_b(movie_hbm, mf_hbm, vals_hbm, pos_hbm, out_hbm,
             midx_v, pidx_v, pos_v, mrows_v, urows_v, out_v, sem):
    wid = lax.axis_index("s") * _NC + lax.axis_index("c")
    base = wid * _BPW

    for j in range(_BPW // 128):
        pltpu.sync_copy(movie_hbm.at[pl.ds(base + j * 128, 128)], midx_v.at[j])
    pltpu.sync_copy(pos_hbm.at[pl.ds(base, _BPW), :], pos_v)

    iota = lax.iota(jnp.int32, L)
    cols = [jnp.full((L,), f, jnp.int32) for f in range(F)]

    for j in range(_BPW // 128):
        def posx(v2, c, j=j):
            rows = (j * 8 + v2) * L + iota
            p = plsc.load_gather(pos_v, [rows, cols[0]])
            pidx_v[j, pl.ds(v2 * L, L)] = p
            return c
        lax.fori_loop(0, 8, posx, 0)

    descs = []
    for j in range(_BPW // 128):
        descs.append(pltpu.async_copy(
            mf_hbm.at[midx_v.at[j]], mrows_v.at[pl.ds(j * 128, 128), :], sem))
        descs.append(pltpu.async_copy(
            vals_hbm.at[pidx_v.at[j]], urows_v.at[pl.ds(j * 128, 128), :],
            sem))
    for d in descs:
        d.wait()

    def blk(v, c):
        rows = v * L + iota
        acc = jnp.zeros((L,), jnp.float32)
        for f in range(F):
            uv = plsc.load_gather(urows_v, [rows, cols[f]])
            mv = plsc.load_gather(mrows_v, [rows, cols[f]])
            acc = acc + uv * mv
        out_v[pl.ds(v * L, L)] = acc
        return c
    lax.fori_loop(0, _BPW // L, blk, 0)

    pltpu.sync_copy(out_v, out_hbm.at[pl.ds(base, _BPW)])


_stage_b_kernel = functools.partial(
    pl.kernel,
    out_type=jax.ShapeDtypeStruct((B,), jnp.float32),
    mesh=plsc.VectorSubcoreMesh(core_axis_name="c", subcore_axis_name="s"),
    compiler_params=pltpu.CompilerParams(
        needs_layout_passes=False, use_tc_tiling_on_sc=False),
    scratch_types=[
        pltpu.VMEM((_BPW // 128, 128), jnp.int32),
        pltpu.VMEM((_BPW // 128, 128), jnp.int32),
        pltpu.VMEM((_BPW, F), jnp.int32),
        pltpu.VMEM((_BPW, F), jnp.float32),
        pltpu.VMEM((_BPW, F), jnp.float32),
        pltpu.VMEM((_BPW,), jnp.float32),
        pltpu.SemaphoreType.DMA,
    ],
)(_stage_b)


def kernel(user, movie, user_factors, movie_factors):
    user = user.astype(jnp.int32)
    movie = movie.astype(jnp.int32)
    vals, slots, counts = _stage_a_kernel(user, user_factors.T)
    pos = _stage_b0_kernel(slots, counts)
    vals2 = vals.reshape(_NW * _CAP, F)
    return _stage_b_kernel(movie, movie_factors, vals2, pos)
